# baseline (speedup 1.0000x reference)
import dataclasses
import functools

import numpy as np
import jax
import jax.numpy as jnp
from jax import lax
from jax.experimental import pallas as pl
from jax.experimental.pallas import tpu as pltpu
from jax.experimental.pallas import tpu_sc as plsc

B, R, C, H, K_KNN, K_SEL, MLP = 4, 16, 64, 4, 10, 4, 256
N = R ** 3
BN = B * N
DH = C // H


def _knn_table():
    g = np.arange(R)
    coords = np.stack(np.meshgrid(g, g, g, indexing="ij"), axis=-1)
    coords = coords.reshape(N, 3).astype(np.int64)
    d2 = ((coords[:, None, :] - coords[None, :, :]) ** 2).sum(-1)
    order = np.argsort(d2, axis=1, kind="stable")[:, :K_KNN]
    kt = order.T.astype(np.int64)
    cols = [kt + b * N for b in range(B)]
    return np.concatenate(cols, axis=1).astype(np.int32)


_KNNT = _knn_table()

_PREC = lax.Precision.DEFAULT


def _dot(a, b):
    return lax.dot_general(a, b, (((1,), (0,)), ((), ())),
                           preferred_element_type=jnp.float32,
                           precision=_PREC)


_TA = 2048


def _stage_a_body(x_ref, n1w_ref, n1b_ref, wq_ref, bq_ref, wkv_ref, ws_ref,
                  q_ref, kv_ref, s_ref):
    x = x_ref[...]
    m = jnp.mean(x, axis=1, keepdims=True)
    v = jnp.mean((x - m) ** 2, axis=1, keepdims=True)
    ln = (x - m) / jnp.sqrt(v + 1e-5) * n1w_ref[...] + n1b_ref[...]
    q_ref[...] = _dot(ln, wq_ref[...]) + bq_ref[...]
    kv = _dot(ln, wkv_ref[...])
    def _rne16(x):
        b = lax.bitcast_convert_type(x, jnp.int32)
        return b + 0x7FFF + jnp.bitwise_and(lax.shift_right_logical(b, 16), 1)
    k16 = lax.shift_right_logical(_rne16(kv[:, :C]), 16)
    v16 = jnp.bitwise_and(_rne16(kv[:, C:]), jnp.int32(-65536))
    kv_ref[...] = jnp.bitwise_or(k16, v16)
    s_ref[...] = jnp.sum(ln * ws_ref[...], axis=1, keepdims=True)


def _stage_a(x, n1w, n1b, wq, bq, wkv, ws):
    nblk = BN // _TA
    full = lambda shape: pl.BlockSpec(shape, lambda i: (0, 0))
    return pl.pallas_call(
        _stage_a_body,
        grid=(nblk,),
        in_specs=[
            pl.BlockSpec((_TA, C), lambda i: (i, 0)),
            full((1, C)), full((1, C)),
            full((C, C)), full((1, C)),
            full((C, 2 * C)), full((1, C)),
        ],
        out_specs=[
            pl.BlockSpec((_TA, C), lambda i: (i, 0)),
            pl.BlockSpec((_TA, C), lambda i: (i, 0)),
            pl.BlockSpec((_TA, 1), lambda i: (i, 0)),
        ],
        out_shape=[
            jax.ShapeDtypeStruct((BN, C), jnp.float32),
            jax.ShapeDtypeStruct((BN, C), jnp.int32),
            jax.ShapeDtypeStruct((BN, 1), jnp.float32),
        ],
    )(x, n1w, n1b, wq, bq, wkv, ws)


_NW = 32
_TPW = BN // _NW
_CH = 32
_NCH = _TPW // _CH
_NGR = _TPW // 16

_NEG = -3.4e38


def _sc_body(s_hbm, knn_hbm, xkv_hbm,
             kv0_hbm, kv1_hbm, kv2_hbm, kv3_hbm, gates_hbm,
             s_v, knn_v, gid_v, rows_v, gates_v, semg, semw):
    kv_outs = (kv0_hbm, kv1_hbm, kv2_hbm, kv3_hbm)
    w = lax.axis_index("s") * 2 + lax.axis_index("c")
    wbase = w * _TPW
    pltpu.sync_copy(s_hbm, s_v)
    pltpu.sync_copy(knn_hbm.at[:, pl.ds(wbase, _TPW)], knn_v)

    lane = lax.iota(jnp.int32, 16)

    @pl.loop(0, _NGR)
    def _(g):
        lb = g * 16
        cand = []
        gids = []
        for k in range(K_KNN):
            idx_k = knn_v[k, pl.ds(lb, 16)]
            gids.append(idx_k)
            cand.append(plsc.load_gather(s_v, [idx_k]))
        for j in range(K_SEL):
            m = cand[0]
            for k in range(1, K_KNN):
                m = jnp.maximum(m, cand[k])
            found = lane < 0
            chosen = gids[0]
            for k in range(K_KNN):
                eq = jnp.logical_and(cand[k] == m, jnp.logical_not(found))
                chosen = jnp.where(eq, gids[k], chosen)
                cand[k] = jnp.where(eq, _NEG, cand[k])
                found = jnp.logical_or(found, eq)
            gate = 1.0 / (1.0 + jnp.exp(-m))
            gid_v[j, pl.ds(lb, 16)] = chosen
            plsc.store_scatter(gates_v, [lb + lane,
                                         jnp.full((16,), j, jnp.int32)],
                               gate)

    gath = [None, None]
    writ = [None, None]

    def fire_gather(c, b):
        gath[b] = [pltpu.async_copy(
            xkv_hbm.at[gid_v.at[j, pl.ds(c * _CH, _CH)]],
            rows_v.at[b, j], semg.at[b]) for j in range(K_SEL)]

    def fire_write(c, b):
        for h in gath[b]:
            h.wait()
        writ[b] = [pltpu.async_copy(
            rows_v.at[b, j], kv_outs[j].at[pl.ds(wbase + c * _CH, _CH)],
            semw.at[b]) for j in range(K_SEL)]

    for c in range(_NCH):
        b = c % 2
        if writ[b] is not None:
            for h in writ[b]:
                h.wait()
            writ[b] = None
        fire_gather(c, b)
        if c >= 1:
            fire_write(c - 1, 1 - b)
    fire_write(_NCH - 1, (_NCH - 1) % 2)
    for b in range(2):
        if writ[b] is not None:
            for h in writ[b]:
                h.wait()

    pltpu.sync_copy(gates_v, gates_hbm.at[pl.ds(wbase, _TPW)])


def _stage_b(s_flat, knn, xkv):
    mesh = plsc.VectorSubcoreMesh(core_axis_name="c", subcore_axis_name="s")
    row = jax.ShapeDtypeStruct((BN, C), jnp.int32)
    cp = pltpu.CompilerParams()
    if "needs_layout_passes" in pltpu.CompilerParams.__dataclass_fields__:
        cp = dataclasses.replace(cp, needs_layout_passes=False)
    if "use_tc_tiling_on_sc" in pltpu.CompilerParams.__dataclass_fields__:
        cp = dataclasses.replace(cp, use_tc_tiling_on_sc=False)
    kern = functools.partial(
        pl.kernel,
        mesh=mesh,
        compiler_params=cp,
        out_type=[row, row, row, row,
                  jax.ShapeDtypeStruct((BN, K_SEL), jnp.float32)],
        scratch_types=[
            pltpu.VMEM((BN,), jnp.float32),
            pltpu.VMEM((K_KNN, _TPW), jnp.int32),
            pltpu.VMEM((K_SEL, _TPW), jnp.int32),
            pltpu.VMEM((2, K_SEL, _CH, C), jnp.int32),
            pltpu.VMEM((_TPW, K_SEL), jnp.float32),
            pltpu.SemaphoreType.DMA((2,)),
            pltpu.SemaphoreType.DMA((2,)),
        ],
    )(_sc_body)
    return kern(s_flat, knn, xkv)


_TC = 1024
_NBC = BN // _TC


def _stage_c_body(q_ref, kv0_ref, kv1_ref, kv2_ref, kv3_ref, g_ref,
                  sc_ref, bk_ref, bv_ref, wo_ref, bo_ref,
                  n2w_ref, n2b_ref, w1_ref, b1_ref, w2_ref, b2_ref,
                  out_ref):
    q = q_ref[...]
    g = g_ref[0]

    def _unpack(r):
        w = r[...]
        xk = lax.bitcast_convert_type(lax.shift_left(w, 16), jnp.float32)
        xv = lax.bitcast_convert_type(
            jnp.bitwise_and(w, jnp.int32(-65536)), jnp.float32)
        return xk, xv

    kv = tuple(_unpack(r) for r in (kv0_ref, kv1_ref, kv2_ref, kv3_ref))

    rows = lax.broadcasted_iota(jnp.int32, (C, H), 0) // DH
    cols = lax.broadcasted_iota(jnp.int32, (C, H), 1)
    S = (rows == cols).astype(jnp.float32)
    rows_t = lax.broadcasted_iota(jnp.int32, (H, C), 0)
    cols_t = lax.broadcasted_iota(jnp.int32, (H, C), 1) // DH
    ST = (rows_t == cols_t).astype(jnp.float32)

    iota4 = lax.broadcasted_iota(jnp.int32, (1, H), 1)
    qbk = _dot(q * bk_ref[...], S)

    scale = jnp.float32(1.0) / jnp.sqrt(jnp.float32(DH))
    g_cols = []
    logits = []
    for j in range(K_SEL):
        g_j = jnp.sum(jnp.where(iota4 == j, g, 0.0), axis=1, keepdims=True)
        g_cols.append(g_j)
        hs = _dot(q * kv[j][0], S)
        logits.append((hs * g_j + qbk) * scale)

    m = jnp.maximum(jnp.maximum(logits[0], logits[1]),
                    jnp.maximum(logits[2], logits[3]))
    es = [jnp.exp(l - m) for l in logits]
    z = es[0] + es[1] + es[2] + es[3]

    out = jnp.zeros_like(q)
    for j in range(K_SEL):
        att_e = _dot(es[j] / z, ST)
        out = out + att_e * (kv[j][1] * g_cols[j] + bv_ref[...])

    o = _dot(out, wo_ref[...]) + bo_ref[...]
    x1 = o * 0.5 + sc_ref[...]

    mu = jnp.mean(x1, axis=1, keepdims=True)
    var = jnp.mean((x1 - mu) ** 2, axis=1, keepdims=True)
    y = (x1 - mu) / jnp.sqrt(var + 1e-5) * n2w_ref[...] + n2b_ref[...]
    h = jax.nn.gelu(_dot(y, w1_ref[...]) + b1_ref[...])
    y2 = _dot(h, w2_ref[...]) + b2_ref[...]
    out_ref[...] = y2 * 0.5 + x1


def _stage_c(q, kvs, gates3, shortcut, bk, bv, wo, bo, n2w, n2b, w1, b1, w2, b2):
    full = lambda shape: pl.BlockSpec(shape, lambda i: tuple(0 for _ in shape))
    tok = lambda width: pl.BlockSpec((_TC, width), lambda i: (i, 0))
    return pl.pallas_call(
        _stage_c_body,
        grid=(_NBC,),
        in_specs=[
            tok(C),
            tok(C), tok(C), tok(C), tok(C),
            pl.BlockSpec((1, _TC, K_SEL), lambda i: (i, 0, 0)),
            tok(C),
            full((1, C)), full((1, C)),
            full((C, C)), full((1, C)),
            full((1, C)), full((1, C)),
            full((C, MLP)), full((1, MLP)),
            full((MLP, C)), full((1, C)),
        ],
        out_specs=pl.BlockSpec((_TC, C), lambda i: (i, 0)),
        out_shape=jax.ShapeDtypeStruct((BN, C), jnp.float32),
    )(q, *kvs, gates3, shortcut, bk, bv, wo, bo, n2w, n2b, w1, b1, w2, b2)


def kernel(inputs, norm1_w, norm1_b, norm2_w, norm2_b, Wq, bq, Wk, bk, Wv, bv,
           Wo, bo, w_score, W1, b1, W2, b2):
    x = inputs.reshape(BN, C)
    wkv = jnp.concatenate([Wk, Wv], axis=1)
    row = lambda a: a.reshape(1, -1)

    q, xkv, s = _stage_a(x, row(norm1_w), row(norm1_b), Wq, row(bq), wkv,
                         row(w_score))

    knn = jnp.asarray(_KNNT)
    kv0, kv1, kv2, kv3, gates = _stage_b(s.reshape(BN), knn, xkv)

    y = _stage_c(q, (kv0, kv1, kv2, kv3), gates.reshape(_NBC, _TC, K_SEL), x,
                 row(bk), row(bv), Wo, row(bo), row(norm2_w), row(norm2_b),
                 W1, row(b1), W2, row(b2))
    return y.reshape(B, N, C)


# speedup vs baseline: 1.3298x; 1.3298x over previous
import dataclasses
import functools

import numpy as np
import jax
import jax.numpy as jnp
from jax import lax
from jax.experimental import pallas as pl
from jax.experimental.pallas import tpu as pltpu
from jax.experimental.pallas import tpu_sc as plsc

B, R, C, H, K_KNN, K_SEL, MLP = 4, 16, 64, 4, 10, 4, 256
N = R ** 3
BN = B * N
DH = C // H


def _knn_table():
    g = np.arange(R)
    coords = np.stack(np.meshgrid(g, g, g, indexing="ij"), axis=-1)
    coords = coords.reshape(N, 3).astype(np.int64)
    d2 = ((coords[:, None, :] - coords[None, :, :]) ** 2).sum(-1)
    order = np.argsort(d2, axis=1, kind="stable")[:, :K_KNN]
    return order.T.astype(np.int32)


_KNNT = _knn_table()

_PREC = lax.Precision.DEFAULT


def _dot(a, b):
    return lax.dot_general(a, b, (((1,), (0,)), ((), ())),
                           preferred_element_type=jnp.float32,
                           precision=_PREC)


_TA = 2048


def _stage_a_body(x_ref, n1w_ref, n1b_ref, wq_ref, bq_ref, wkv_ref, ws_ref,
                  q_ref, kv_ref, s_ref):
    x = x_ref[...]
    m = jnp.mean(x, axis=1, keepdims=True)
    v = jnp.mean((x - m) ** 2, axis=1, keepdims=True)
    ln = (x - m) / jnp.sqrt(v + 1e-5) * n1w_ref[...] + n1b_ref[...]
    q_ref[...] = _dot(ln, wq_ref[...]) + bq_ref[...]
    kv = _dot(ln, wkv_ref[...])
    def _rne16(x):
        b = lax.bitcast_convert_type(x, jnp.int32)
        return b + 0x7FFF + jnp.bitwise_and(lax.shift_right_logical(b, 16), 1)
    k16 = lax.shift_right_logical(_rne16(kv[:, :C]), 16)
    v16 = jnp.bitwise_and(_rne16(kv[:, C:]), jnp.int32(-65536))
    kvw = jnp.bitwise_or(k16, v16)
    h = _TA // 4
    kv_ref[...] = jnp.concatenate(
        [jnp.concatenate([kvw[0:h], kvw[h:2 * h]], axis=1),
         jnp.concatenate([kvw[2 * h:3 * h], kvw[3 * h:]], axis=1)], axis=0)
    s = jnp.sum(ln * ws_ref[...], axis=1, keepdims=True)
    s_ref[...] = s.reshape(_TA // 128, 128)


def _stage_a(x, n1w, n1b, wq, bq, wkv, ws):
    nblk = BN // _TA
    full = lambda shape: pl.BlockSpec(shape, lambda i: (0, 0))
    return pl.pallas_call(
        _stage_a_body,
        grid=(nblk,),
        in_specs=[
            pl.BlockSpec((_TA, C), lambda i: (i, 0)),
            full((1, C)), full((1, C)),
            full((C, C)), full((1, C)),
            full((C, 2 * C)), full((1, C)),
        ],
        out_specs=[
            pl.BlockSpec((_TA, C), lambda i: (i, 0)),
            pl.BlockSpec((_TA // 2, 2 * C), lambda i: (i, 0)),
            pl.BlockSpec((_TA // 128, 128), lambda i: (i, 0)),
        ],
        out_shape=[
            jax.ShapeDtypeStruct((BN, C), jnp.float32),
            jax.ShapeDtypeStruct((BN // 2, 2 * C), jnp.int32),
            jax.ShapeDtypeStruct((BN // 128, 128), jnp.float32),
        ],
    )(x, n1w, n1b, wq, bq, wkv, ws)


_NW = 32
_TPW = BN // _NW
_CH = 32
_NCH = _TPW // _CH
_NGR = _TPW // 16

_NEG = -3.4e38


def _sc_body(s_hbm, knn_hbm, xkv_hbm,
             kv0_hbm, kv1_hbm, kv2_hbm, kv3_hbm, gates_hbm,
             s_v, knn_v, gid_v, rows_v, gates_v, semg, semw):
    kv_outs = (kv0_hbm, kv1_hbm, kv2_hbm, kv3_hbm)
    w = lax.axis_index("s") * 2 + lax.axis_index("c")
    wbase = w * _TPW
    bidx = w // 8
    wloc = (w % 8) * _TPW
    rbase = (w // 2) * _TPW
    half = w % 2
    pltpu.sync_copy(s_hbm.at[pl.ds(bidx * (N // 128), N // 128), :], s_v)
    pltpu.sync_copy(knn_hbm.at[:, pl.ds(wloc, _TPW)], knn_v)

    lane = lax.iota(jnp.int32, 16)

    @pl.loop(0, _NGR)
    def _(g):
        lb = g * 16
        cand = []
        gids = []
        for k in range(K_KNN):
            idx_k = knn_v[k, pl.ds(lb, 16)]
            gids.append(idx_k)
            cand.append(plsc.load_gather(
                s_v, [lax.shift_right_logical(idx_k, 7),
                      jnp.bitwise_and(idx_k, 127)]))
        for j in range(K_SEL):
            m = cand[0]
            for k in range(1, K_KNN):
                m = jnp.maximum(m, cand[k])
            found = lane < 0
            chosen = gids[0]
            for k in range(K_KNN):
                eq = jnp.logical_and(cand[k] == m, jnp.logical_not(found))
                chosen = jnp.where(eq, gids[k], chosen)
                cand[k] = jnp.where(eq, _NEG, cand[k])
                found = jnp.logical_or(found, eq)
            gate = 1.0 / (1.0 + jnp.exp(-m))
            row = ((chosen >> 10) << 10) | ((chosen & 511) << 1) \
                | ((chosen & 1023) >> 9)
            gid_v[j, pl.ds(lb, 16)] = row + bidx * N
            plsc.store_scatter(gates_v, [jnp.full((16,), j, jnp.int32),
                                         lb + lane], gate)

    gath = [None, None]
    writ = [None, None]

    def fire_gather(c, b):
        gath[b] = [pltpu.async_copy(
            xkv_hbm.at[gid_v.at[j, pl.ds(c * _CH, _CH)]],
            rows_v.at[b, j], semg.at[b]) for j in range(K_SEL)]

    def fire_write(c, b):
        for h in gath[b]:
            h.wait()
        writ[b] = [pltpu.async_copy(
            rows_v.at[b, j],
            kv_outs[j].at[pl.ds(rbase + c * _CH, _CH),
                          pl.ds(half * C, C)],
            semw.at[b]) for j in range(K_SEL)]

    for c in range(_NCH):
        b = c % 2
        if writ[b] is not None:
            for h in writ[b]:
                h.wait()
            writ[b] = None
        fire_gather(c, b)
        if c >= 1:
            fire_write(c - 1, 1 - b)
    fire_write(_NCH - 1, (_NCH - 1) % 2)
    for b in range(2):
        if writ[b] is not None:
            for h in writ[b]:
                h.wait()

    pltpu.sync_copy(gates_v,
                    gates_hbm.at[pl.ds(0, K_SEL), pl.ds(wbase, _TPW)])


def _stage_b(s_flat, knn, xkv):
    mesh = plsc.VectorSubcoreMesh(core_axis_name="c", subcore_axis_name="s")
    row = jax.ShapeDtypeStruct((BN // 2, 2 * C), jnp.int32)
    cp = pltpu.CompilerParams()
    if "needs_layout_passes" in pltpu.CompilerParams.__dataclass_fields__:
        cp = dataclasses.replace(cp, needs_layout_passes=False)
    if "use_tc_tiling_on_sc" in pltpu.CompilerParams.__dataclass_fields__:
        cp = dataclasses.replace(cp, use_tc_tiling_on_sc=False)
    kern = functools.partial(
        pl.kernel,
        mesh=mesh,
        compiler_params=cp,
        out_type=[row, row, row, row,
                  jax.ShapeDtypeStruct((8, BN), jnp.float32)],
        scratch_types=[
            pltpu.VMEM((N // 128, 128), jnp.float32),
            pltpu.VMEM((K_KNN, _TPW), jnp.int32),
            pltpu.VMEM((K_SEL, _TPW), jnp.int32),
            pltpu.VMEM((2, K_SEL, _CH, C), jnp.int32),
            pltpu.VMEM((K_SEL, _TPW), jnp.float32),
            pltpu.SemaphoreType.DMA((2,)),
            pltpu.SemaphoreType.DMA((2,)),
        ],
    )(_sc_body)
    return kern(s_flat, knn, xkv)


_TC = 1024
_NBC = BN // _TC


def _stage_c_body(q_ref, kv0_ref, kv1_ref, kv2_ref, kv3_ref, g_ref,
                  sc_ref, bk_ref, bv_ref, wo_ref, bo_ref,
                  n2w_ref, n2b_ref, w1_ref, b1_ref, w2_ref, b2_ref,
                  out_ref):
    T2 = _TC // 2
    qq = q_ref[...]
    gt = jnp.transpose(g_ref[...])
    kvw = [r[...] for r in (kv0_ref, kv1_ref, kv2_ref, kv3_ref)]

    rows = lax.broadcasted_iota(jnp.int32, (C, H), 0) // DH
    cols = lax.broadcasted_iota(jnp.int32, (C, H), 1)
    S = (rows == cols).astype(jnp.float32)
    rows_t = lax.broadcasted_iota(jnp.int32, (H, C), 0)
    cols_t = lax.broadcasted_iota(jnp.int32, (H, C), 1) // DH
    ST = (rows_t == cols_t).astype(jnp.float32)
    iota8 = lax.broadcasted_iota(jnp.int32, (1, 8), 1)
    scale = jnp.float32(1.0) / jnp.sqrt(jnp.float32(DH))
    w2b = w2_ref[...].astype(jnp.bfloat16)

    for hh in range(2):
        sl = slice(hh * T2, (hh + 1) * T2)
        q = qq[sl]
        g = gt[sl]
        qbk = _dot(q * bk_ref[...], S)

        g_cols = []
        logits = []
        for j in range(K_SEL):
            w = kvw[j][:, hh * C:(hh + 1) * C]
            xk = lax.bitcast_convert_type(lax.shift_left(w, 16), jnp.float32)
            g_j = jnp.sum(jnp.where(iota8 == j, g, 0.0), axis=1,
                          keepdims=True)
            g_cols.append(g_j)
            hs = _dot(q * xk, S)
            logits.append((hs * g_j + qbk) * scale)

        m = jnp.maximum(jnp.maximum(logits[0], logits[1]),
                        jnp.maximum(logits[2], logits[3]))
        es = [jnp.exp(l - m) for l in logits]
        z = es[0] + es[1] + es[2] + es[3]

        out = jnp.zeros_like(q)
        for j in range(K_SEL):
            w = kvw[j][:, hh * C:(hh + 1) * C]
            xv = lax.bitcast_convert_type(
                jnp.bitwise_and(w, jnp.int32(-65536)), jnp.float32)
            att_e = _dot(es[j] / z, ST)
            out = out + att_e * (xv * g_cols[j] + bv_ref[...])

        o = _dot(out, wo_ref[...]) + bo_ref[...]
        x1 = o * 0.5 + sc_ref[sl]

        mu = jnp.mean(x1, axis=1, keepdims=True)
        var = jnp.mean((x1 - mu) ** 2, axis=1, keepdims=True)
        y = (x1 - mu) / jnp.sqrt(var + 1e-5) * n2w_ref[...] + n2b_ref[...]
        hb = jax.nn.gelu((_dot(y, w1_ref[...]) + b1_ref[...])
                         .astype(jnp.bfloat16))
        y2 = _dot(hb, w2b) + b2_ref[...]
        out_ref[sl] = y2 * 0.5 + x1


def _stage_c(q, kvs, gates, shortcut, bk, bv, wo, bo, n2w, n2b, w1, b1, w2, b2):
    full = lambda shape: pl.BlockSpec(shape, lambda i: tuple(0 for _ in shape))
    tok = lambda width: pl.BlockSpec((_TC, width), lambda i: (i, 0))
    return pl.pallas_call(
        _stage_c_body,
        grid=(_NBC,),
        in_specs=[
            tok(C),
            pl.BlockSpec((_TC // 2, 2 * C), lambda i: (i, 0)),
            pl.BlockSpec((_TC // 2, 2 * C), lambda i: (i, 0)),
            pl.BlockSpec((_TC // 2, 2 * C), lambda i: (i, 0)),
            pl.BlockSpec((_TC // 2, 2 * C), lambda i: (i, 0)),
            pl.BlockSpec((8, _TC), lambda i: (0, i)),
            tok(C),
            full((1, C)), full((1, C)),
            full((C, C)), full((1, C)),
            full((1, C)), full((1, C)),
            full((C, MLP)), full((1, MLP)),
            full((MLP, C)), full((1, C)),
        ],
        out_specs=pl.BlockSpec((_TC, C), lambda i: (i, 0)),
        out_shape=jax.ShapeDtypeStruct((BN, C), jnp.float32),
    )(q, *kvs, gates, shortcut, bk, bv, wo, bo, n2w, n2b, w1, b1, w2, b2)


def kernel(inputs, norm1_w, norm1_b, norm2_w, norm2_b, Wq, bq, Wk, bk, Wv, bv,
           Wo, bo, w_score, W1, b1, W2, b2):
    x = inputs.reshape(BN, C)
    wkv = jnp.concatenate([Wk, Wv], axis=1)
    row = lambda a: a.reshape(1, -1)

    q, pkv, s2 = _stage_a(x, row(norm1_w), row(norm1_b), Wq, row(bq), wkv,
                          row(w_score))

    knn = jnp.asarray(_KNNT)
    kv0, kv1, kv2, kv3, gates = _stage_b(s2, knn, pkv.reshape(BN, C))

    y = _stage_c(q, (kv0, kv1, kv2, kv3), gates, x,
                 row(bk), row(bv), Wo, row(bo), row(norm2_w), row(norm2_b),
                 W1, row(b1), W2, row(b2))
    return y.reshape(B, N, C)


# speedup vs baseline: 1.3666x; 1.0276x over previous
import dataclasses
import functools

import numpy as np
import jax
import jax.numpy as jnp
from jax import lax
from jax.experimental import pallas as pl
from jax.experimental.pallas import tpu as pltpu
from jax.experimental.pallas import tpu_sc as plsc

B, R, C, H, K_KNN, K_SEL, MLP = 4, 16, 64, 4, 10, 4, 256
N = R ** 3
BN = B * N
DH = C // H


def _knn_table():
    g = np.arange(R)
    coords = np.stack(np.meshgrid(g, g, g, indexing="ij"), axis=-1)
    coords = coords.reshape(N, 3).astype(np.int64)
    d2 = ((coords[:, None, :] - coords[None, :, :]) ** 2).sum(-1)
    order = np.argsort(d2, axis=1, kind="stable")[:, :K_KNN]
    return order.T.astype(np.int32)


_KNNT = _knn_table()

_PREC = lax.Precision.DEFAULT


def _dot(a, b):
    return lax.dot_general(a, b, (((1,), (0,)), ((), ())),
                           preferred_element_type=jnp.float32,
                           precision=_PREC)


_TA = 2048


def _stage_a_body(x_ref, n1w_ref, n1b_ref, wq_ref, bq_ref, wkv_ref, ws_ref,
                  q_ref, kv_ref, s_ref):
    x = x_ref[0]
    m = jnp.mean(x, axis=1, keepdims=True)
    v = jnp.mean((x - m) ** 2, axis=1, keepdims=True)
    ln = (x - m) / jnp.sqrt(v + 1e-5) * n1w_ref[...] + n1b_ref[...]
    q_ref[...] = _dot(ln, wq_ref[...]) + bq_ref[...]
    kv = _dot(ln, wkv_ref[...])
    def _rne16(x):
        b = lax.bitcast_convert_type(x, jnp.int32)
        return b + 0x7FFF + jnp.bitwise_and(lax.shift_right_logical(b, 16), 1)
    k16 = lax.shift_right_logical(_rne16(kv[:, :C]), 16)
    v16 = jnp.bitwise_and(_rne16(kv[:, C:]), jnp.int32(-65536))
    kvw = jnp.bitwise_or(k16, v16)
    h = _TA // 4
    kv_ref[...] = jnp.concatenate(
        [jnp.concatenate([kvw[0:h], kvw[h:2 * h]], axis=1),
         jnp.concatenate([kvw[2 * h:3 * h], kvw[3 * h:]], axis=1)], axis=0)
    s = jnp.sum(ln * ws_ref[...], axis=1, keepdims=True)
    s_ref[...] = s.reshape(_TA // 128, 128)


def _stage_a(x, n1w, n1b, wq, bq, wkv, ws):
    nblk = BN // _TA
    full = lambda shape: pl.BlockSpec(shape, lambda i: (0, 0))
    return pl.pallas_call(
        _stage_a_body,
        grid=(nblk,),
        in_specs=[
            pl.BlockSpec((1, _TA, C), lambda i: (i // 2, i % 2, 0)),
            full((1, C)), full((1, C)),
            full((C, C)), full((1, C)),
            full((C, 2 * C)), full((1, C)),
        ],
        out_specs=[
            pl.BlockSpec((_TA, C), lambda i: (i, 0)),
            pl.BlockSpec((_TA // 2, 2 * C), lambda i: (i, 0)),
            pl.BlockSpec((_TA // 128, 128), lambda i: (i, 0)),
        ],
        out_shape=[
            jax.ShapeDtypeStruct((BN, C), jnp.float32),
            jax.ShapeDtypeStruct((BN // 2, 2 * C), jnp.int32),
            jax.ShapeDtypeStruct((BN // 128, 128), jnp.float32),
        ],
    )(x, n1w, n1b, wq, bq, wkv, ws)


_NW = 32
_TPW = BN // _NW
_CH = 64
_NCH = _TPW // _CH
_NGR = _TPW // 16

_NEG = -3.4e38


def _sc_body(s_hbm, knn_hbm, xkv_hbm,
             kv0_hbm, kv1_hbm, kv2_hbm, kv3_hbm, gates_hbm,
             s_v, knn_v, gid_v, rows_v, gates_v, semg, semw):
    kv_outs = (kv0_hbm, kv1_hbm, kv2_hbm, kv3_hbm)
    w = lax.axis_index("s") * 2 + lax.axis_index("c")
    wbase = w * _TPW
    bidx = w // 8
    wloc = (w % 8) * _TPW
    rbase = (w // 2) * _TPW
    half = w % 2
    pltpu.sync_copy(s_hbm.at[pl.ds(bidx * (N // 128), N // 128), :], s_v)
    pltpu.sync_copy(knn_hbm.at[:, pl.ds(wloc, _TPW)], knn_v)

    lane = lax.iota(jnp.int32, 16)

    def select_chunk(c):
        @pl.loop(0, _CH // 16)
        def _(g):
            lb = c * _CH + g * 16
            cand = []
            gids = []
            for k in range(K_KNN):
                idx_k = knn_v[k, pl.ds(lb, 16)]
                gids.append(idx_k)
                cand.append(plsc.load_gather(
                    s_v, [lax.shift_right_logical(idx_k, 7),
                          jnp.bitwise_and(idx_k, 127)]))
            for j in range(K_SEL):
                m = cand[0]
                for k in range(1, K_KNN):
                    m = jnp.maximum(m, cand[k])
                found = lane < 0
                chosen = gids[0]
                for k in range(K_KNN):
                    eq = jnp.logical_and(cand[k] == m,
                                         jnp.logical_not(found))
                    chosen = jnp.where(eq, gids[k], chosen)
                    cand[k] = jnp.where(eq, _NEG, cand[k])
                    found = jnp.logical_or(found, eq)
                gate = 1.0 / (1.0 + jnp.exp(-m))
                row = ((chosen >> 10) << 10) | ((chosen & 511) << 1) \
                    | ((chosen & 1023) >> 9)
                gid_v[j, pl.ds(lb, 16)] = row + bidx * N
                plsc.store_scatter(gates_v, [jnp.full((16,), j, jnp.int32),
                                             lb + lane], gate)

    gath = [None, None]
    writ = [None, None]

    def fire_gather(c, b):
        gath[b] = [pltpu.async_copy(
            xkv_hbm.at[gid_v.at[j, pl.ds(c * _CH, _CH)]],
            rows_v.at[b, j], semg.at[b]) for j in range(K_SEL)]

    def fire_write(c, b):
        for h in gath[b]:
            h.wait()
        writ[b] = [pltpu.async_copy(
            rows_v.at[b, j],
            kv_outs[j].at[pl.ds(rbase + c * _CH, _CH),
                          pl.ds(half * C, C)],
            semw.at[b]) for j in range(K_SEL)]

    for c in range(_NCH):
        b = c % 2
        select_chunk(c)
        if writ[b] is not None:
            for h in writ[b]:
                h.wait()
            writ[b] = None
        fire_gather(c, b)
        if c >= 1:
            fire_write(c - 1, 1 - b)
    fire_write(_NCH - 1, (_NCH - 1) % 2)
    for b in range(2):
        if writ[b] is not None:
            for h in writ[b]:
                h.wait()

    pltpu.sync_copy(gates_v,
                    gates_hbm.at[pl.ds(0, K_SEL), pl.ds(wbase, _TPW)])


def _stage_b(s_flat, knn, xkv):
    mesh = plsc.VectorSubcoreMesh(core_axis_name="c", subcore_axis_name="s")
    row = jax.ShapeDtypeStruct((BN // 2, 2 * C), jnp.int32)
    cp = pltpu.CompilerParams()
    if "needs_layout_passes" in pltpu.CompilerParams.__dataclass_fields__:
        cp = dataclasses.replace(cp, needs_layout_passes=False)
    if "use_tc_tiling_on_sc" in pltpu.CompilerParams.__dataclass_fields__:
        cp = dataclasses.replace(cp, use_tc_tiling_on_sc=False)
    kern = functools.partial(
        pl.kernel,
        mesh=mesh,
        compiler_params=cp,
        out_type=[row, row, row, row,
                  jax.ShapeDtypeStruct((8, BN), jnp.float32)],
        scratch_types=[
            pltpu.VMEM((N // 128, 128), jnp.float32),
            pltpu.VMEM((K_KNN, _TPW), jnp.int32),
            pltpu.VMEM((K_SEL, _TPW), jnp.int32),
            pltpu.VMEM((2, K_SEL, _CH, C), jnp.int32),
            pltpu.VMEM((K_SEL, _TPW), jnp.float32),
            pltpu.SemaphoreType.DMA((2,)),
            pltpu.SemaphoreType.DMA((2,)),
        ],
    )(_sc_body)
    return kern(s_flat, knn, xkv)


_TC = 1024
_NBC = BN // _TC


def _stage_c_body(q_ref, kv0_ref, kv1_ref, kv2_ref, kv3_ref, g_ref,
                  sc_ref, bk_ref, bv_ref, wo_ref, bo_ref,
                  n2w_ref, n2b_ref, w1_ref, b1_ref, w2_ref, b2_ref,
                  out_ref):
    T2 = _TC // 2
    qq = q_ref[...]
    gt = jnp.transpose(g_ref[...])
    kvw = [r[...] for r in (kv0_ref, kv1_ref, kv2_ref, kv3_ref)]

    rows = lax.broadcasted_iota(jnp.int32, (C, H), 0) // DH
    cols = lax.broadcasted_iota(jnp.int32, (C, H), 1)
    S = (rows == cols).astype(jnp.float32)
    rows_t = lax.broadcasted_iota(jnp.int32, (H, C), 0)
    cols_t = lax.broadcasted_iota(jnp.int32, (H, C), 1) // DH
    ST = (rows_t == cols_t).astype(jnp.float32)
    iota8 = lax.broadcasted_iota(jnp.int32, (1, 8), 1)
    scale = jnp.float32(1.0) / jnp.sqrt(jnp.float32(DH))
    w2b = w2_ref[...].astype(jnp.bfloat16)

    for hh in range(2):
        sl = slice(hh * T2, (hh + 1) * T2)
        q = qq[sl]
        g = gt[sl]
        qbk = _dot(q * bk_ref[...], S)

        g_cols = []
        logits = []
        for j in range(K_SEL):
            w = kvw[j][:, hh * C:(hh + 1) * C]
            xk = lax.bitcast_convert_type(lax.shift_left(w, 16), jnp.float32)
            g_j = jnp.sum(jnp.where(iota8 == j, g, 0.0), axis=1,
                          keepdims=True)
            g_cols.append(g_j)
            hs = _dot(q * xk, S)
            logits.append((hs * g_j + qbk) * scale)

        m = jnp.maximum(jnp.maximum(logits[0], logits[1]),
                        jnp.maximum(logits[2], logits[3]))
        es = [jnp.exp(l - m) for l in logits]
        z = es[0] + es[1] + es[2] + es[3]

        out = jnp.zeros_like(q)
        for j in range(K_SEL):
            w = kvw[j][:, hh * C:(hh + 1) * C]
            xv = lax.bitcast_convert_type(
                jnp.bitwise_and(w, jnp.int32(-65536)), jnp.float32)
            att_e = _dot(es[j] / z, ST)
            out = out + att_e * (xv * g_cols[j] + bv_ref[...])

        o = _dot(out, wo_ref[...]) + bo_ref[...]
        x1 = o * 0.5 + sc_ref[0, sl]

        mu = jnp.mean(x1, axis=1, keepdims=True)
        var = jnp.mean((x1 - mu) ** 2, axis=1, keepdims=True)
        y = (x1 - mu) / jnp.sqrt(var + 1e-5) * n2w_ref[...] + n2b_ref[...]
        hb = jax.nn.gelu((_dot(y, w1_ref[...]) + b1_ref[...])
                         .astype(jnp.bfloat16))
        y2 = _dot(hb, w2b) + b2_ref[...]
        out_ref[0, sl] = y2 * 0.5 + x1


def _stage_c(q, kvs, gates, shortcut, bk, bv, wo, bo, n2w, n2b, w1, b1, w2, b2):
    full = lambda shape: pl.BlockSpec(shape, lambda i: tuple(0 for _ in shape))
    tok = lambda width: pl.BlockSpec((_TC, width), lambda i: (i, 0))
    return pl.pallas_call(
        _stage_c_body,
        grid=(_NBC,),
        in_specs=[
            tok(C),
            pl.BlockSpec((_TC // 2, 2 * C), lambda i: (i, 0)),
            pl.BlockSpec((_TC // 2, 2 * C), lambda i: (i, 0)),
            pl.BlockSpec((_TC // 2, 2 * C), lambda i: (i, 0)),
            pl.BlockSpec((_TC // 2, 2 * C), lambda i: (i, 0)),
            pl.BlockSpec((8, _TC), lambda i: (0, i)),
            pl.BlockSpec((1, _TC, C), lambda i: (i // 4, i % 4, 0)),
            full((1, C)), full((1, C)),
            full((C, C)), full((1, C)),
            full((1, C)), full((1, C)),
            full((C, MLP)), full((1, MLP)),
            full((MLP, C)), full((1, C)),
        ],
        out_specs=pl.BlockSpec((1, _TC, C), lambda i: (i // 4, i % 4, 0)),
        out_shape=jax.ShapeDtypeStruct((B, N, C), jnp.float32),
    )(q, *kvs, gates, shortcut, bk, bv, wo, bo, n2w, n2b, w1, b1, w2, b2)


def kernel(inputs, norm1_w, norm1_b, norm2_w, norm2_b, Wq, bq, Wk, bk, Wv, bv,
           Wo, bo, w_score, W1, b1, W2, b2):
    wkv = jnp.concatenate([Wk, Wv], axis=1)
    row = lambda a: a.reshape(1, -1)

    q, pkv, s2 = _stage_a(inputs, row(norm1_w), row(norm1_b), Wq, row(bq),
                          wkv, row(w_score))

    knn = jnp.asarray(_KNNT)
    kv0, kv1, kv2, kv3, gates = _stage_b(s2, knn, pkv.reshape(BN, C))

    return _stage_c(q, (kv0, kv1, kv2, kv3), gates, inputs,
                    row(bk), row(bv), Wo, row(bo), row(norm2_w),
                    row(norm2_b), W1, row(b1), W2, row(b2))


# speedup vs baseline: 1.6449x; 1.2037x over previous
import dataclasses
import functools

import numpy as np
import jax
import jax.numpy as jnp
from jax import lax
from jax.experimental import pallas as pl
from jax.experimental.pallas import tpu as pltpu
from jax.experimental.pallas import tpu_sc as plsc

B, R, C, H, K_KNN, K_SEL, MLP = 4, 16, 64, 4, 10, 4, 256
N = R ** 3
BN = B * N
DH = C // H


def _knn_table():
    g = np.arange(R)
    coords = np.stack(np.meshgrid(g, g, g, indexing="ij"), axis=-1)
    coords = coords.reshape(N, 3).astype(np.int64)
    d2 = ((coords[:, None, :] - coords[None, :, :]) ** 2).sum(-1)
    order = np.argsort(d2, axis=1, kind="stable")[:, :K_KNN]
    return order.T.astype(np.int32)


_KNN_LOCAL = _knn_table()


def _knn_worker_major():
    parts = [_KNN_LOCAL[:, g * 512:(g + 1) * 512].reshape(-1)
             for g in range(N // 512)]
    return np.concatenate(parts).astype(np.int32)


_KNNT = _knn_worker_major()

_PREC = lax.Precision.DEFAULT


def _dot(a, b):
    return lax.dot_general(a, b, (((1,), (0,)), ((), ())),
                           preferred_element_type=jnp.float32,
                           precision=_PREC)


def _dotg(a, b, ca, cb):
    return lax.dot_general(a, b, (((ca,), (cb,)), ((), ())),
                           preferred_element_type=jnp.float32,
                           precision=_PREC)


_TA = 2048


def _stage_a_body(x_ref, n1w_ref, n1b_ref, wq_ref, bq_ref, wk_ref, wv_ref,
                  ws_ref, q_ref, kv_ref, s_ref):
    xt = x_ref[0]
    m = jnp.mean(xt, axis=0, keepdims=True)
    v = jnp.mean((xt - m) ** 2, axis=0, keepdims=True)
    lnt = (xt - m) / jnp.sqrt(v + 1e-5) * n1w_ref[...] + n1b_ref[...]
    q_ref[...] = _dotg(lnt, wq_ref[...], 0, 0) + bq_ref[...]
    xk = _dotg(lnt, wk_ref[...], 0, 0)
    xv = _dotg(lnt, wv_ref[...], 0, 0)
    def _rne16(x):
        b = lax.bitcast_convert_type(x, jnp.int32)
        return b + 0x7FFF + jnp.bitwise_and(lax.shift_right_logical(b, 16), 1)
    k16 = lax.shift_right_logical(_rne16(xk), 16)
    v16 = jnp.bitwise_and(_rne16(xv), jnp.int32(-65536))
    kvw = jnp.bitwise_or(k16, v16)
    h = _TA // 4
    kv_ref[...] = jnp.concatenate(
        [jnp.concatenate([kvw[0:h], kvw[h:2 * h]], axis=1),
         jnp.concatenate([kvw[2 * h:3 * h], kvw[3 * h:]], axis=1)], axis=0)
    s_ref[...] = _dot(ws_ref[...], lnt).reshape(_TA // 128, 128)


def _stage_a(xt, n1w, n1b, wq, bq, wk, wv, ws):
    nblk = BN // _TA
    full = lambda shape: pl.BlockSpec(shape, lambda i: (0, 0))
    return pl.pallas_call(
        _stage_a_body,
        grid=(nblk,),
        in_specs=[
            pl.BlockSpec((1, C, _TA), lambda i: (i // 2, 0, i % 2)),
            full((C, 1)), full((C, 1)),
            full((C, C)), full((1, C)),
            full((C, C)), full((C, C)),
            full((1, C)),
        ],
        out_specs=[
            pl.BlockSpec((_TA, C), lambda i: (i, 0)),
            pl.BlockSpec((_TA // 2, 2 * C), lambda i: (i, 0)),
            pl.BlockSpec((_TA // 128, 128), lambda i: (i, 0)),
        ],
        out_shape=[
            jax.ShapeDtypeStruct((BN, C), jnp.float32),
            jax.ShapeDtypeStruct((BN // 2, 2 * C), jnp.int32),
            jax.ShapeDtypeStruct((BN // 128, 128), jnp.float32),
        ],
    )(xt, n1w, n1b, wq, bq, wk, wv, ws)


_NW = 32
_TPW = BN // _NW
_CH = 64
_NCH = _TPW // _CH
_NGR = _TPW // 16

_NEG = -3.4e38


def _sc_body(s_hbm, knn_hbm, xkv_hbm,
             kv0_hbm, kv1_hbm, kv2_hbm, kv3_hbm, gates_hbm,
             s_v, knn_v, gid_v, rows_v, gates_v, semg, semw):
    kv_outs = (kv0_hbm, kv1_hbm, kv2_hbm, kv3_hbm)
    w = lax.axis_index("s") * 2 + lax.axis_index("c")
    wbase = w * _TPW
    bidx = w // 8
    wloc = (w % 8) * _TPW
    rbase = (w // 2) * _TPW
    half = w % 2
    pltpu.sync_copy(s_hbm.at[pl.ds(bidx * 32, 32), :], s_v)
    pltpu.sync_copy(knn_hbm.at[pl.ds((w % 8) * (K_KNN * _TPW),
                                     K_KNN * _TPW)], knn_v)

    lane = lax.iota(jnp.int32, 16)

    def select_chunk(c):
        @pl.loop(0, _CH // 16)
        def _(g):
            lb = c * _CH + g * 16
            cand = []
            gids = []
            for k in range(K_KNN):
                idx_k = knn_v[pl.ds(k * _TPW + lb, 16)]
                gids.append(idx_k)
                cand.append(plsc.load_gather(
                    s_v, [lax.shift_right_logical(idx_k, 7),
                          jnp.bitwise_and(idx_k, 127)]))
            for j in range(K_SEL):
                m = cand[0]
                for k in range(1, K_KNN):
                    m = jnp.maximum(m, cand[k])
                found = lane < 0
                chosen = gids[0]
                for k in range(K_KNN):
                    eq = jnp.logical_and(cand[k] == m,
                                         jnp.logical_not(found))
                    chosen = jnp.where(eq, gids[k], chosen)
                    cand[k] = jnp.where(eq, _NEG, cand[k])
                    found = jnp.logical_or(found, eq)
                gate = 1.0 / (1.0 + jnp.exp(-m))
                row = ((chosen >> 10) << 10) | ((chosen & 511) << 1) \
                    | ((chosen & 1023) >> 9)
                gid_v[j, pl.ds(lb, 16)] = row + bidx * N
                lt = lb + lane
                plsc.store_scatter(
                    gates_v, [lax.shift_right_logical(lt, 7),
                              jnp.full((16,), j, jnp.int32),
                              jnp.bitwise_and(lt, 127)], gate)

    gath = [None, None]
    writ = [None, None]

    def fire_gather(c, b):
        gath[b] = [pltpu.async_copy(
            xkv_hbm.at[gid_v.at[j, pl.ds(c * _CH, _CH)]],
            rows_v.at[b, j], semg.at[b]) for j in range(K_SEL)]

    def fire_write(c, b):
        for h in gath[b]:
            h.wait()
        writ[b] = [pltpu.async_copy(
            rows_v.at[b, j],
            kv_outs[j].at[pl.ds(rbase + c * _CH, _CH),
                          pl.ds(half * C, C)],
            semw.at[b]) for j in range(K_SEL)]

    for c in range(_NCH):
        b = c % 2
        select_chunk(c)
        if writ[b] is not None:
            for h in writ[b]:
                h.wait()
            writ[b] = None
        fire_gather(c, b)
        if c >= 1:
            fire_write(c - 1, 1 - b)
    fire_write(_NCH - 1, (_NCH - 1) % 2)
    for b in range(2):
        if writ[b] is not None:
            for h in writ[b]:
                h.wait()

    pltpu.sync_copy(gates_v, gates_hbm.at[pl.ds(w * 4, 4)])


def _stage_b(s_flat, knn, xkv):
    mesh = plsc.VectorSubcoreMesh(core_axis_name="c", subcore_axis_name="s")
    row = jax.ShapeDtypeStruct((BN // 2, 2 * C), jnp.int32)
    cp = pltpu.CompilerParams()
    if "needs_layout_passes" in pltpu.CompilerParams.__dataclass_fields__:
        cp = dataclasses.replace(cp, needs_layout_passes=False)
    if "use_tc_tiling_on_sc" in pltpu.CompilerParams.__dataclass_fields__:
        cp = dataclasses.replace(cp, use_tc_tiling_on_sc=False)
    kern = functools.partial(
        pl.kernel,
        mesh=mesh,
        compiler_params=cp,
        out_type=[row, row, row, row,
                  jax.ShapeDtypeStruct((BN // 128, 8, 128), jnp.float32)],
        scratch_types=[
            pltpu.VMEM((32, 128), jnp.float32),
            pltpu.VMEM((K_KNN * _TPW,), jnp.int32),
            pltpu.VMEM((K_SEL, _TPW), jnp.int32),
            pltpu.VMEM((2, K_SEL, _CH, C), jnp.int32),
            pltpu.VMEM((4, 8, 128), jnp.float32),
            pltpu.SemaphoreType.DMA((2,)),
            pltpu.SemaphoreType.DMA((2,)),
        ],
    )(_sc_body)
    return kern(s_flat, knn, xkv)


_TC = 1024
_NBC = BN // _TC


def _stage_c_body(q_ref, kv0_ref, kv1_ref, kv2_ref, kv3_ref, g_ref,
                  sc_ref, bk_ref, bv_ref, wo_ref, bo_ref,
                  n2w_ref, n2b_ref, w1_ref, b1_ref, w2_ref, b2_ref,
                  out_ref):
    T2 = _TC // 2
    qq = q_ref[...]
    gt = jnp.transpose(g_ref[...])
    kvw = [r[...] for r in (kv0_ref, kv1_ref, kv2_ref, kv3_ref)]

    rows = lax.broadcasted_iota(jnp.int32, (C, H), 0) // DH
    cols = lax.broadcasted_iota(jnp.int32, (C, H), 1)
    S = (rows == cols).astype(jnp.float32)
    rows_t = lax.broadcasted_iota(jnp.int32, (H, C), 0)
    cols_t = lax.broadcasted_iota(jnp.int32, (H, C), 1) // DH
    ST = (rows_t == cols_t).astype(jnp.float32)
    iota8 = lax.broadcasted_iota(jnp.int32, (1, 8), 1)
    scale = jnp.float32(1.0) / jnp.sqrt(jnp.float32(DH))
    w2b = w2_ref[...].astype(jnp.bfloat16)

    for hh in range(2):
        sl = slice(hh * T2, (hh + 1) * T2)
        q = qq[sl]
        g = gt[sl]
        qbk = _dot(q * bk_ref[...], S)

        g_cols = []
        logits = []
        for j in range(K_SEL):
            w = kvw[j][:, hh * C:(hh + 1) * C]
            xk = lax.bitcast_convert_type(lax.shift_left(w, 16), jnp.float32)
            g_j = jnp.sum(jnp.where(iota8 == j, g, 0.0), axis=1,
                          keepdims=True)
            g_cols.append(g_j)
            hs = _dot(q * xk, S)
            logits.append((hs * g_j + qbk) * scale)

        m = jnp.maximum(jnp.maximum(logits[0], logits[1]),
                        jnp.maximum(logits[2], logits[3]))
        es = [jnp.exp(l - m) for l in logits]
        z = es[0] + es[1] + es[2] + es[3]

        out = jnp.zeros_like(q)
        for j in range(K_SEL):
            w = kvw[j][:, hh * C:(hh + 1) * C]
            xv = lax.bitcast_convert_type(
                jnp.bitwise_and(w, jnp.int32(-65536)), jnp.float32)
            att_e = _dot(es[j] / z, ST)
            out = out + att_e * (xv * g_cols[j] + bv_ref[...])

        ot = _dotg(wo_ref[...], out, 0, 1) + bo_ref[...]
        x1 = ot * 0.5 + sc_ref[0][:, sl]

        mu = jnp.mean(x1, axis=0, keepdims=True)
        var = jnp.mean((x1 - mu) ** 2, axis=0, keepdims=True)
        y = (x1 - mu) / jnp.sqrt(var + 1e-5) * n2w_ref[...] + n2b_ref[...]
        hb = jax.nn.gelu((_dotg(w1_ref[...], y, 0, 0) + b1_ref[...])
                         .astype(jnp.bfloat16))
        y2 = _dotg(w2b, hb, 0, 0) + b2_ref[...]
        out_ref[0, :, sl] = y2 * 0.5 + x1


def _stage_c(q, kvs, gates, shortcut, bk, bv, wo, bo, n2w, n2b, w1, b1, w2, b2):
    full = lambda shape: pl.BlockSpec(shape, lambda i: tuple(0 for _ in shape))
    tok = lambda width: pl.BlockSpec((_TC, width), lambda i: (i, 0))
    return pl.pallas_call(
        _stage_c_body,
        grid=(_NBC,),
        in_specs=[
            tok(C),
            pl.BlockSpec((_TC // 2, 2 * C), lambda i: (i, 0)),
            pl.BlockSpec((_TC // 2, 2 * C), lambda i: (i, 0)),
            pl.BlockSpec((_TC // 2, 2 * C), lambda i: (i, 0)),
            pl.BlockSpec((_TC // 2, 2 * C), lambda i: (i, 0)),
            pl.BlockSpec((8, _TC), lambda i: (0, i)),
            pl.BlockSpec((1, C, _TC), lambda i: (i // 4, 0, i % 4)),
            full((1, C)), full((1, C)),
            full((C, C)), full((C, 1)),
            full((C, 1)), full((C, 1)),
            full((C, MLP)), full((MLP, 1)),
            full((MLP, C)), full((C, 1)),
        ],
        out_specs=pl.BlockSpec((1, C, _TC), lambda i: (i // 4, 0, i % 4)),
        out_shape=jax.ShapeDtypeStruct((B, C, N), jnp.float32),
    )(q, *kvs, gates, shortcut, bk, bv, wo, bo, n2w, n2b, w1, b1, w2, b2)


def kernel(inputs, norm1_w, norm1_b, norm2_w, norm2_b, Wq, bq, Wk, bk, Wv, bv,
           Wo, bo, w_score, W1, b1, W2, b2):
    row = lambda a: a.reshape(1, -1)
    col = lambda a: a.reshape(-1, 1)
    xt = jnp.transpose(inputs, (0, 2, 1))

    q, pkv, s2 = _stage_a(xt, col(norm1_w), col(norm1_b), Wq, row(bq),
                          Wk, Wv, row(w_score))

    knn = jnp.asarray(_KNNT)
    kv0, kv1, kv2, kv3, g3 = _stage_b(s2, knn, pkv.reshape(BN, C))
    gates = jnp.swapaxes(g3, 0, 1).reshape(8, BN)

    yt = _stage_c(q, (kv0, kv1, kv2, kv3), gates, xt,
                  row(bk), row(bv), Wo, col(bo), col(norm2_w),
                  col(norm2_b), W1, col(b1), W2, col(b2))
    return jnp.transpose(yt, (0, 2, 1))


# speedup vs baseline: 1.7069x; 1.0377x over previous
import dataclasses
import functools

import numpy as np
import jax
import jax.numpy as jnp
from jax import lax
from jax.experimental import pallas as pl
from jax.experimental.pallas import tpu as pltpu
from jax.experimental.pallas import tpu_sc as plsc

B, R, C, H, K_KNN, K_SEL, MLP = 4, 16, 64, 4, 10, 4, 256
N = R ** 3
BN = B * N
DH = C // H


def _knn_table():
    g = np.arange(R)
    coords = np.stack(np.meshgrid(g, g, g, indexing="ij"), axis=-1)
    coords = coords.reshape(N, 3).astype(np.int64)
    d2 = ((coords[:, None, :] - coords[None, :, :]) ** 2).sum(-1)
    order = np.argsort(d2, axis=1, kind="stable")[:, :K_KNN]
    return order.T.astype(np.int32)


_KNN_LOCAL = _knn_table()


def _knn_worker_major():
    parts = [_KNN_LOCAL[:, g * 512:(g + 1) * 512].reshape(-1)
             for g in range(N // 512)]
    return np.concatenate(parts).astype(np.int32)


_KNNT = _knn_worker_major()

_PREC = lax.Precision.DEFAULT


def _dot(a, b):
    return lax.dot_general(a, b, (((1,), (0,)), ((), ())),
                           preferred_element_type=jnp.float32,
                           precision=_PREC)


def _dotg(a, b, ca, cb):
    return lax.dot_general(a, b, (((ca,), (cb,)), ((), ())),
                           preferred_element_type=jnp.float32,
                           precision=_PREC)


_TA = 4096


def _stage_a_body(x_ref, n1w_ref, n1b_ref, wq_ref, bq_ref, wk_ref, wv_ref,
                  ws_ref, q_ref, kv_ref, s_ref):
    xt = x_ref[0]
    m = jnp.mean(xt, axis=0, keepdims=True)
    v = jnp.mean((xt - m) ** 2, axis=0, keepdims=True)
    lnt = (xt - m) / jnp.sqrt(v + 1e-5) * n1w_ref[...] + n1b_ref[...]
    q_ref[...] = _dotg(lnt, wq_ref[...], 0, 0) + bq_ref[...]
    xk = _dotg(lnt, wk_ref[...], 0, 0)
    xv = _dotg(lnt, wv_ref[...], 0, 0)
    def _rne16(x):
        b = lax.bitcast_convert_type(x, jnp.int32)
        return b + 0x7FFF + jnp.bitwise_and(lax.shift_right_logical(b, 16), 1)
    k16 = lax.shift_right_logical(_rne16(xk), 16)
    v16 = jnp.bitwise_and(_rne16(xv), jnp.int32(-65536))
    kvw = jnp.bitwise_or(k16, v16)
    kv_ref[...] = jnp.concatenate(
        [jnp.concatenate([kvw[g * 1024:g * 1024 + 512],
                          kvw[g * 1024 + 512:(g + 1) * 1024]], axis=1)
         for g in range(_TA // 1024)], axis=0)
    s_ref[...] = _dot(ws_ref[...], lnt).reshape(_TA // 128, 128)


def _stage_a(xt, n1w, n1b, wq, bq, wk, wv, ws):
    nblk = BN // _TA
    full = lambda shape: pl.BlockSpec(shape, lambda i: (0, 0))
    return pl.pallas_call(
        _stage_a_body,
        grid=(nblk,),
        in_specs=[
            pl.BlockSpec((1, C, _TA), lambda i: (i, 0, 0)),
            full((C, 1)), full((C, 1)),
            full((C, C)), full((1, C)),
            full((C, C)), full((C, C)),
            full((1, C)),
        ],
        out_specs=[
            pl.BlockSpec((_TA, C), lambda i: (i, 0)),
            pl.BlockSpec((_TA // 2, 2 * C), lambda i: (i, 0)),
            pl.BlockSpec((_TA // 128, 128), lambda i: (i, 0)),
        ],
        out_shape=[
            jax.ShapeDtypeStruct((BN, C), jnp.float32),
            jax.ShapeDtypeStruct((BN // 2, 2 * C), jnp.int32),
            jax.ShapeDtypeStruct((BN // 128, 128), jnp.float32),
        ],
    )(xt, n1w, n1b, wq, bq, wk, wv, ws)


_NW = 32
_TPW = BN // _NW
_CH = 64
_NCH = _TPW // _CH
_NGR = _TPW // 16

_NEG = -3.4e38


def _sc_body(s_hbm, knn_hbm, xkv_hbm,
             kv0_hbm, kv1_hbm, kv2_hbm, kv3_hbm, gates_hbm,
             s_v, knn_v, gid_v, rows_v, gates_v, semg, semw):
    kv_outs = (kv0_hbm, kv1_hbm, kv2_hbm, kv3_hbm)
    w = lax.axis_index("s") * 2 + lax.axis_index("c")
    wbase = w * _TPW
    bidx = w // 8
    wloc = (w % 8) * _TPW
    rbase = (w // 2) * _TPW
    half = w % 2
    pltpu.sync_copy(s_hbm.at[pl.ds(bidx * 32, 32), :], s_v)
    pltpu.sync_copy(knn_hbm.at[pl.ds((w % 8) * (K_KNN * _TPW),
                                     K_KNN * _TPW)], knn_v)

    lane = lax.iota(jnp.int32, 16)

    def select_chunk(c):
        @pl.loop(0, _CH // 16)
        def _(g):
            lb = c * _CH + g * 16
            cand = []
            gids = []
            for k in range(K_KNN):
                idx_k = knn_v[pl.ds(k * _TPW + lb, 16)]
                gids.append(idx_k)
                cand.append(plsc.load_gather(
                    s_v, [lax.shift_right_logical(idx_k, 7),
                          jnp.bitwise_and(idx_k, 127)]))
            for j in range(K_SEL):
                m = cand[0]
                for k in range(1, K_KNN):
                    m = jnp.maximum(m, cand[k])
                found = lane < 0
                chosen = gids[0]
                for k in range(K_KNN):
                    eq = jnp.logical_and(cand[k] == m,
                                         jnp.logical_not(found))
                    chosen = jnp.where(eq, gids[k], chosen)
                    cand[k] = jnp.where(eq, _NEG, cand[k])
                    found = jnp.logical_or(found, eq)
                gate = 1.0 / (1.0 + jnp.exp(-m))
                row = ((chosen >> 10) << 10) | ((chosen & 511) << 1) \
                    | ((chosen & 1023) >> 9)
                gid_v[j, pl.ds(lb, 16)] = row + bidx * N
                lt = lb + lane
                plsc.store_scatter(
                    gates_v, [lax.shift_right_logical(lt, 7),
                              jnp.full((16,), j, jnp.int32),
                              jnp.bitwise_and(lt, 127)], gate)

    gath = [None, None]
    writ = [None, None]

    def fire_gather(c, b):
        gath[b] = [pltpu.async_copy(
            xkv_hbm.at[gid_v.at[j, pl.ds(c * _CH, _CH)]],
            rows_v.at[b, j], semg.at[b]) for j in range(K_SEL)]

    def fire_write(c, b):
        for h in gath[b]:
            h.wait()
        writ[b] = [pltpu.async_copy(
            rows_v.at[b, j],
            kv_outs[j].at[pl.ds(rbase + c * _CH, _CH),
                          pl.ds(half * C, C)],
            semw.at[b]) for j in range(K_SEL)]

    for c in range(_NCH):
        b = c % 2
        select_chunk(c)
        if writ[b] is not None:
            for h in writ[b]:
                h.wait()
            writ[b] = None
        fire_gather(c, b)
        if c >= 1:
            fire_write(c - 1, 1 - b)
    fire_write(_NCH - 1, (_NCH - 1) % 2)
    for b in range(2):
        if writ[b] is not None:
            for h in writ[b]:
                h.wait()

    pltpu.sync_copy(gates_v, gates_hbm.at[pl.ds(w * 4, 4)])


def _stage_b(s_flat, knn, xkv):
    mesh = plsc.VectorSubcoreMesh(core_axis_name="c", subcore_axis_name="s")
    row = jax.ShapeDtypeStruct((BN // 2, 2 * C), jnp.int32)
    cp = pltpu.CompilerParams()
    if "needs_layout_passes" in pltpu.CompilerParams.__dataclass_fields__:
        cp = dataclasses.replace(cp, needs_layout_passes=False)
    if "use_tc_tiling_on_sc" in pltpu.CompilerParams.__dataclass_fields__:
        cp = dataclasses.replace(cp, use_tc_tiling_on_sc=False)
    kern = functools.partial(
        pl.kernel,
        mesh=mesh,
        compiler_params=cp,
        out_type=[row, row, row, row,
                  jax.ShapeDtypeStruct((BN // 128, 8, 128), jnp.float32)],
        scratch_types=[
            pltpu.VMEM((32, 128), jnp.float32),
            pltpu.VMEM((K_KNN * _TPW,), jnp.int32),
            pltpu.VMEM((K_SEL, _TPW), jnp.int32),
            pltpu.VMEM((2, K_SEL, _CH, C), jnp.int32),
            pltpu.VMEM((4, 8, 128), jnp.float32),
            pltpu.SemaphoreType.DMA((2,)),
            pltpu.SemaphoreType.DMA((2,)),
        ],
    )(_sc_body)
    return kern(s_flat, knn, xkv)


_TC = 2048
_NBC = BN // _TC


def _stage_c_body(q_ref, kv0_ref, kv1_ref, kv2_ref, kv3_ref, g_ref,
                  sc_ref, bk_ref, bv_ref, wo_ref, bo_ref,
                  n2w_ref, n2b_ref, w1_ref, b1_ref, w2_ref, b2_ref,
                  out_ref):
    T2 = 512
    qq = q_ref[...]
    gt = jnp.transpose(g_ref[...])
    kvw = [r[...] for r in (kv0_ref, kv1_ref, kv2_ref, kv3_ref)]

    rows = lax.broadcasted_iota(jnp.int32, (C, H), 0) // DH
    cols = lax.broadcasted_iota(jnp.int32, (C, H), 1)
    S = (rows == cols).astype(jnp.float32)
    rows_t = lax.broadcasted_iota(jnp.int32, (H, C), 0)
    cols_t = lax.broadcasted_iota(jnp.int32, (H, C), 1) // DH
    ST = (rows_t == cols_t).astype(jnp.float32)
    iota8 = lax.broadcasted_iota(jnp.int32, (1, 8), 1)
    scale = jnp.float32(1.0) / jnp.sqrt(jnp.float32(DH))
    w2b = w2_ref[...].astype(jnp.bfloat16)

    for part in range(_TC // 512):
        g10, hh = divmod(part, 2)
        sl = slice(g10 * 1024 + hh * 512, g10 * 1024 + hh * 512 + 512)
        rsl = slice(g10 * 512, (g10 + 1) * 512)
        q = qq[sl]
        g = gt[sl]
        qbk = _dot(q * bk_ref[...], S)

        g_cols = []
        logits = []
        for j in range(K_SEL):
            w = kvw[j][rsl, hh * C:(hh + 1) * C]
            xk = lax.bitcast_convert_type(lax.shift_left(w, 16), jnp.float32)
            g_j = jnp.sum(jnp.where(iota8 == j, g, 0.0), axis=1,
                          keepdims=True)
            g_cols.append(g_j)
            hs = _dot(q * xk, S)
            logits.append((hs * g_j + qbk) * scale)

        m = jnp.maximum(jnp.maximum(logits[0], logits[1]),
                        jnp.maximum(logits[2], logits[3]))
        es = [jnp.exp(l - m) for l in logits]
        z = es[0] + es[1] + es[2] + es[3]

        out = jnp.zeros_like(q)
        for j in range(K_SEL):
            w = kvw[j][rsl, hh * C:(hh + 1) * C]
            xv = lax.bitcast_convert_type(
                jnp.bitwise_and(w, jnp.int32(-65536)), jnp.float32)
            att_e = _dot(es[j] / z, ST)
            out = out + att_e * (xv * g_cols[j] + bv_ref[...])

        ot = _dotg(wo_ref[...], out, 0, 1) + bo_ref[...]
        x1 = ot * 0.5 + sc_ref[0][:, sl]

        mu = jnp.mean(x1, axis=0, keepdims=True)
        var = jnp.mean((x1 - mu) ** 2, axis=0, keepdims=True)
        y = (x1 - mu) / jnp.sqrt(var + 1e-5) * n2w_ref[...] + n2b_ref[...]
        hb = jax.nn.gelu((_dotg(w1_ref[...], y, 0, 0) + b1_ref[...])
                         .astype(jnp.bfloat16))
        y2 = _dotg(w2b, hb, 0, 0) + b2_ref[...]
        out_ref[0, :, sl] = y2 * 0.5 + x1


def _stage_c(q, kvs, gates, shortcut, bk, bv, wo, bo, n2w, n2b, w1, b1, w2, b2):
    full = lambda shape: pl.BlockSpec(shape, lambda i: tuple(0 for _ in shape))
    tok = lambda width: pl.BlockSpec((_TC, width), lambda i: (i, 0))
    return pl.pallas_call(
        _stage_c_body,
        grid=(_NBC,),
        in_specs=[
            tok(C),
            pl.BlockSpec((_TC // 2, 2 * C), lambda i: (i, 0)),
            pl.BlockSpec((_TC // 2, 2 * C), lambda i: (i, 0)),
            pl.BlockSpec((_TC // 2, 2 * C), lambda i: (i, 0)),
            pl.BlockSpec((_TC // 2, 2 * C), lambda i: (i, 0)),
            pl.BlockSpec((8, _TC), lambda i: (0, i)),
            pl.BlockSpec((1, C, _TC), lambda i: (i // 2, 0, i % 2)),
            full((1, C)), full((1, C)),
            full((C, C)), full((C, 1)),
            full((C, 1)), full((C, 1)),
            full((C, MLP)), full((MLP, 1)),
            full((MLP, C)), full((C, 1)),
        ],
        out_specs=pl.BlockSpec((1, C, _TC), lambda i: (i // 2, 0, i % 2)),
        out_shape=jax.ShapeDtypeStruct((B, C, N), jnp.float32),
    )(q, *kvs, gates, shortcut, bk, bv, wo, bo, n2w, n2b, w1, b1, w2, b2)


def kernel(inputs, norm1_w, norm1_b, norm2_w, norm2_b, Wq, bq, Wk, bk, Wv, bv,
           Wo, bo, w_score, W1, b1, W2, b2):
    row = lambda a: a.reshape(1, -1)
    col = lambda a: a.reshape(-1, 1)
    xt = jnp.transpose(inputs, (0, 2, 1))

    q, pkv, s2 = _stage_a(xt, col(norm1_w), col(norm1_b), Wq, row(bq),
                          Wk, Wv, row(w_score))

    knn = jnp.asarray(_KNNT)
    kv0, kv1, kv2, kv3, g3 = _stage_b(s2, knn, pkv.reshape(BN, C))
    gates = jnp.swapaxes(g3, 0, 1).reshape(8, BN)

    yt = _stage_c(q, (kv0, kv1, kv2, kv3), gates, xt,
                  row(bk), row(bv), Wo, col(bo), col(norm2_w),
                  col(norm2_b), W1, col(b1), W2, col(b2))
    return jnp.transpose(yt, (0, 2, 1))
